# Initial kernel scaffold; baseline (speedup 1.0000x reference)
import jax
import jax.numpy as jnp
from jax.experimental import pallas as pl


def kernel(mem, mem_label, x, y, idx, retrieve_idx):
    raise NotImplementedError("write your pallas kernel here")


# speedup vs baseline: 1.2116x; 1.2116x over previous
import functools

import jax
import jax.numpy as jnp
from jax import lax
from jax.experimental import pallas as pl
from jax.experimental.pallas import tpu as pltpu
from jax.experimental.pallas import tpu_sc as plsc

M, D, B, NCLS = 100000, 512, 16384, 1000

NC, NS = 2, 16
NW = NC * NS
SLOTS = 3136
LAST = M - (NW - 1) * SLOTS
IB = 8192
NSEG = B // IB
assert SLOTS % 32 == 0 and LAST % 32 == 0 and LAST > 0 and B % IB == 0
_mesh = plsc.VectorSubcoreMesh(core_axis_name="c", subcore_axis_name="s")


def _i32(v):
    return jnp.asarray(v, jnp.int32)


@functools.partial(
    pl.kernel,
    out_type=[
        jax.ShapeDtypeStruct((B, D), jnp.float32),
        jax.ShapeDtypeStruct((B,), jnp.int32),
        jax.ShapeDtypeStruct((M, D), jnp.float32),
        jax.ShapeDtypeStruct((M,), jnp.int32),
    ],
    mesh=_mesh,
    compiler_params=pltpu.CompilerParams(needs_layout_passes=False),
    scratch_types=[
        pltpu.VMEM((IB,), jnp.int32),
        pltpu.VMEM((B,), jnp.int32),
        pltpu.VMEM((SLOTS,), jnp.int32),
        pltpu.VMEM((SLOTS + 48,), jnp.int32),
        pltpu.VMEM((SLOTS + 48,), jnp.int32),
        pltpu.VMEM((IB + 48,), jnp.int32),
        pltpu.VMEM((IB + 48,), jnp.int32),
        pltpu.VMEM((IB + 48,), jnp.int32),
        pltpu.VMEM((IB + 48,), jnp.int32),
        pltpu.VMEM((32, D), jnp.float32),
        pltpu.VMEM((32,), jnp.int32),
        pltpu.VMEM((32, D), jnp.float32),
        pltpu.VMEM((SLOTS,), jnp.int32),
        pltpu.SemaphoreType.DMA,
        pltpu.SemaphoreType.DMA,
        pltpu.SemaphoreType.DMA,
        pltpu.SemaphoreType.DMA,
        pltpu.SemaphoreType.DMA,
        pltpu.SemaphoreType.DMA,
        pltpu.SemaphoreType.DMA,
        pltpu.SemaphoreType.DMA,
        pltpu.SemaphoreType.DMA,
        pltpu.SemaphoreType.DMA,
    ],
)
def _sc_kernel(mem, mem_label, x, y, idx, retrieve_idx,
               rx, ry, nm, nl,
               ibuf, ybuf, poslocal, wslot, wpos, uj, ur, wj, wp,
               rowbuf, lbuf, cbuf, lcbuf,
               sem_g0, sem_g1, sem_s0, sem_s1, sem_l0, sem_l1,
               sem_ia, sem_ib, sem_oa, sem_ob):
    cid = lax.axis_index("c")
    sid = lax.axis_index("s")
    wid = sid * NC + cid
    lo = wid * SLOTS
    is_last = wid == NW - 1
    myslots = jnp.where(is_last, _i32(LAST), _i32(SLOTS))
    hi = lo + myslots
    iota = lax.iota(jnp.int32, 16)
    zeros = iota * 0

    def bcast_at(ref, base):
        return plsc.load_gather(ref, [zeros + base])

    pltpu.sync_copy(y, ybuf)
    neg1 = jnp.full((16,), -1, jnp.int32)

    def init_body(v, carry):
        poslocal[pl.ds(v * 16, 16)] = neg1
        return carry
    lax.fori_loop(0, SLOTS // 16, init_body, 0, unroll=4)

    for seg in range(NSEG):
        pltpu.sync_copy(idx.at[pl.ds(seg * IB, IB)], ibuf)

        def a_body(v, carry):
            idxv = ibuf[pl.ds(v * 16, 16)]
            posv = seg * IB + v * 16 + iota
            inr = (idxv >= lo) & (idxv < hi)
            tgt = jnp.clip(idxv - lo, 0, SLOTS - 1)
            plsc.store_scatter(poslocal, [tgt], posv, mask=inr)

            def w_cond(it):
                cur = plsc.load_gather(poslocal, [tgt])
                return jnp.any(inr & (cur < posv))

            def w_body(it):
                cur = plsc.load_gather(poslocal, [tgt])
                plsc.store_scatter(poslocal, [tgt], posv,
                                   mask=inr & (cur < posv))
                return it + 1
            lax.while_loop(w_cond, w_body, 0)
            return carry
        lax.fori_loop(0, IB // 16, a_body, 0)

    nch2 = myslots // 32

    def in_cp(chunk, half):
        return pltpu.async_copy(
            mem.at[pl.ds(lo + chunk * 16, 16)],
            cbuf.at[pl.ds(half * 16, 16)],
            sem_ia if half == 0 else sem_ib)

    def out_cp(chunk, half):
        return pltpu.async_copy(
            cbuf.at[pl.ds(half * 16, 16)],
            nm.at[pl.ds(lo + chunk * 16, 16)],
            sem_oa if half == 0 else sem_ob)

    in_cp(0, 0)

    def d_body(k, carry):
        ca = 2 * k
        cb = 2 * k + 1
        in_cp(cb, 1)
        pltpu.make_async_copy(mem.at[pl.ds(lo + ca * 16, 16)],
                              cbuf.at[pl.ds(0, 16)], sem_ia).wait()
        oa = out_cp(ca, 0)
        pltpu.make_async_copy(mem.at[pl.ds(lo + cb * 16, 16)],
                              cbuf.at[pl.ds(16, 16)], sem_ib).wait()
        ob = out_cp(cb, 1)
        oa.wait()

        @pl.when(k + 1 < nch2)
        def _():
            in_cp(2 * (k + 1), 0)
        ob.wait()
        return carry
    lax.fori_loop(0, nch2, d_body, 0)

    @pl.when(jnp.logical_not(is_last))
    def _():
        pltpu.sync_copy(mem_label.at[pl.ds(lo, SLOTS)], lcbuf)
        pltpu.sync_copy(lcbuf, nl.at[pl.ds(lo, SLOTS)])

    @pl.when(is_last)
    def _():
        pltpu.sync_copy(mem_label.at[pl.ds(lo, LAST)], lcbuf.at[pl.ds(0, LAST)])
        pltpu.sync_copy(lcbuf.at[pl.ds(0, LAST)], nl.at[pl.ds(lo, LAST)])

    def move_rows(cnt, dstl, srcl, src_hbm, dstrow, dstlbl, local_vals):
        @pl.when(cnt > 0)
        def _():
            dstl[pl.ds(cnt, 16)] = bcast_at(dstl, cnt - 1)
            srcl[pl.ds(cnt, 16)] = bcast_at(srcl, cnt - 1)
            nch = (cnt + 15) // 16
            npair = (nch + 1) // 2

            def prep(c):
                bb = c * 16
                dv = dstl[pl.ds(bb, 16)]
                sv = srcl[pl.ds(bb, 16)]
                valid = (bb + iota) < cnt
                return (jnp.where(valid, dv, bcast_at(dstl, bb)),
                        jnp.where(valid, sv, bcast_at(srcl, bb)))

            def vals(sv):
                if local_vals:
                    return plsc.load_gather(lcbuf, [sv - lo])
                return plsc.load_gather(ybuf, [sv])

            def chunk(c):
                dv, sv = prep(c)
                pltpu.async_copy(src_hbm.at[sv],
                                 rowbuf.at[pl.ds(0, 16)], sem_g0).wait()
                pltpu.async_copy(rowbuf.at[pl.ds(0, 16)],
                                 dstrow.at[dv], sem_s0).wait()
                lbuf[pl.ds(0, 16)] = vals(sv)
                pltpu.async_copy(lbuf.at[pl.ds(0, 16)],
                                 dstlbl.at[dv], sem_l0).wait()

            def pair(k, carry):
                chunk(2 * k)
                chunk(2 * k + 1)
                return carry
            lax.fori_loop(0, npair, pair, 0)

    def b_scan(v, cnt):
        pv = poslocal[pl.ds(v * 16, 16)]
        mask = pv >= 0
        slot_abs = lo + v * 16 + iota
        plsc.store_compressed(wslot.at[pl.ds(cnt, 16)], slot_abs, mask=mask)
        plsc.store_compressed(wpos.at[pl.ds(cnt, 16)], pv, mask=mask)
        return cnt + jnp.sum(mask.astype(jnp.int32))
    wcnt = lax.fori_loop(0, myslots // 16, b_scan, _i32(0))
    move_rows(wcnt, wslot, wpos, x, nm, nl, local_vals=False)

    for seg in range(NSEG):
        pltpu.sync_copy(retrieve_idx.at[pl.ds(seg * IB, IB)], ibuf)

        def c_scan(v, carry):
            ucnt, vcnt = carry
            rv = ibuf[pl.ds(v * 16, 16)]
            inr = (rv >= lo) & (rv < hi)
            rloc = jnp.clip(rv - lo, 0, SLOTS - 1)
            pvals = plsc.load_gather(poslocal, [rloc])
            wr = inr & (pvals >= 0)
            un = inr & (pvals < 0)
            j = seg * IB + v * 16 + iota
            plsc.store_compressed(uj.at[pl.ds(ucnt, 16)], j, mask=un)
            plsc.store_compressed(ur.at[pl.ds(ucnt, 16)], rv, mask=un)
            plsc.store_compressed(wj.at[pl.ds(vcnt, 16)], j, mask=wr)
            plsc.store_compressed(wp.at[pl.ds(vcnt, 16)], pvals, mask=wr)
            return (ucnt + jnp.sum(un.astype(jnp.int32)),
                    vcnt + jnp.sum(wr.astype(jnp.int32)))
        ucnt, vcnt = lax.fori_loop(0, IB // 16, c_scan, (_i32(0), _i32(0)))
        move_rows(ucnt, uj, ur, mem, rx, ry, local_vals=True)
        move_rows(vcnt, wj, wp, x, rx, ry, local_vals=False)


def kernel(mem, mem_label, x, y, idx, retrieve_idx):
    return tuple(_sc_kernel(mem, mem_label, x, y, idx, retrieve_idx))


# speedup vs baseline: 1.2610x; 1.0408x over previous
import functools

import jax
import jax.numpy as jnp
from jax import lax
from jax.experimental import pallas as pl
from jax.experimental.pallas import tpu as pltpu
from jax.experimental.pallas import tpu_sc as plsc

M, D, B, NCLS = 100000, 512, 16384, 1000

NC, NS = 2, 16
NW = NC * NS
SLOTS = 3136
LAST = M - (NW - 1) * SLOTS
IB = 8192
NSEG = B // IB
assert SLOTS % 32 == 0 and LAST % 32 == 0 and LAST > 0 and B % IB == 0
_mesh = plsc.VectorSubcoreMesh(core_axis_name="c", subcore_axis_name="s")


def _i32(v):
    return jnp.asarray(v, jnp.int32)


@functools.partial(
    pl.kernel,
    out_type=[
        jax.ShapeDtypeStruct((B, D), jnp.float32),
        jax.ShapeDtypeStruct((B,), jnp.int32),
        jax.ShapeDtypeStruct((M, D), jnp.float32),
        jax.ShapeDtypeStruct((M,), jnp.int32),
    ],
    mesh=_mesh,
    compiler_params=pltpu.CompilerParams(needs_layout_passes=False),
    scratch_types=[
        pltpu.VMEM((IB,), jnp.int32),
        pltpu.VMEM((B,), jnp.int32),
        pltpu.VMEM((SLOTS,), jnp.int32),
        pltpu.VMEM((SLOTS + 48,), jnp.int32),
        pltpu.VMEM((SLOTS + 48,), jnp.int32),
        pltpu.VMEM((IB + 48,), jnp.int32),
        pltpu.VMEM((IB + 48,), jnp.int32),
        pltpu.VMEM((IB + 48,), jnp.int32),
        pltpu.VMEM((IB + 48,), jnp.int32),
        pltpu.VMEM((32, D), jnp.float32),
        pltpu.VMEM((32,), jnp.int32),
        pltpu.VMEM((32, D), jnp.float32),
        pltpu.VMEM((SLOTS,), jnp.int32),
        pltpu.VMEM((4, 16), jnp.int32),
        pltpu.SemaphoreType.DMA,
        pltpu.SemaphoreType.DMA,
        pltpu.SemaphoreType.DMA,
        pltpu.SemaphoreType.DMA,
        pltpu.SemaphoreType.DMA,
        pltpu.SemaphoreType.DMA,
        pltpu.SemaphoreType.DMA,
        pltpu.SemaphoreType.DMA,
        pltpu.SemaphoreType.DMA,
        pltpu.SemaphoreType.DMA,
    ],
)
def _sc_kernel(mem, mem_label, x, y, idx, retrieve_idx,
               rx, ry, nm, nl,
               ibuf, ybuf, poslocal, wslot, wpos, uj, ur, wj, wp,
               rowbuf, lbuf, cbuf, lcbuf, idxs,
               sem_g0, sem_g1, sem_s0, sem_s1, sem_l0, sem_l1,
               sem_ia, sem_ib, sem_oa, sem_ob):
    cid = lax.axis_index("c")
    sid = lax.axis_index("s")
    wid = sid * NC + cid
    lo = wid * SLOTS
    is_last = wid == NW - 1
    myslots = jnp.where(is_last, _i32(LAST), _i32(SLOTS))
    hi = lo + myslots
    iota = lax.iota(jnp.int32, 16)
    zeros = iota * 0

    def bcast_at(ref, base):
        return plsc.load_gather(ref, [zeros + base])

    pltpu.sync_copy(y, ybuf)
    neg1 = jnp.full((16,), -1, jnp.int32)

    def init_body(v, carry):
        poslocal[pl.ds(v * 16, 16)] = neg1
        return carry
    lax.fori_loop(0, SLOTS // 16, init_body, 0, unroll=4)

    for seg in range(NSEG):
        pltpu.sync_copy(idx.at[pl.ds(seg * IB, IB)], ibuf)

        def a_body(v, carry):
            idxv = ibuf[pl.ds(v * 16, 16)]
            posv = seg * IB + v * 16 + iota
            inr = (idxv >= lo) & (idxv < hi)
            tgt = jnp.clip(idxv - lo, 0, SLOTS - 1)
            plsc.store_scatter(poslocal, [tgt], posv, mask=inr)

            def w_cond(it):
                cur = plsc.load_gather(poslocal, [tgt])
                return jnp.any(inr & (cur < posv))

            def w_body(it):
                cur = plsc.load_gather(poslocal, [tgt])
                plsc.store_scatter(poslocal, [tgt], posv,
                                   mask=inr & (cur < posv))
                return it + 1
            lax.while_loop(w_cond, w_body, 0)
            return carry
        lax.fori_loop(0, IB // 16, a_body, 0)

    nch2 = myslots // 32

    def in_cp(chunk, half):
        return pltpu.async_copy(
            mem.at[pl.ds(lo + chunk * 16, 16)],
            cbuf.at[pl.ds(half * 16, 16)],
            sem_ia if half == 0 else sem_ib)

    def out_cp(chunk, half):
        return pltpu.async_copy(
            cbuf.at[pl.ds(half * 16, 16)],
            nm.at[pl.ds(lo + chunk * 16, 16)],
            sem_oa if half == 0 else sem_ob)

    in_cp(0, 0)

    def d_body(k, carry):
        ca = 2 * k
        cb = 2 * k + 1
        in_cp(cb, 1)
        pltpu.make_async_copy(mem.at[pl.ds(lo + ca * 16, 16)],
                              cbuf.at[pl.ds(0, 16)], sem_ia).wait()
        oa = out_cp(ca, 0)
        pltpu.make_async_copy(mem.at[pl.ds(lo + cb * 16, 16)],
                              cbuf.at[pl.ds(16, 16)], sem_ib).wait()
        ob = out_cp(cb, 1)
        oa.wait()

        @pl.when(k + 1 < nch2)
        def _():
            in_cp(2 * (k + 1), 0)
        ob.wait()
        return carry
    lax.fori_loop(0, nch2, d_body, 0)

    @pl.when(jnp.logical_not(is_last))
    def _():
        pltpu.sync_copy(mem_label.at[pl.ds(lo, SLOTS)], lcbuf)
        pltpu.sync_copy(lcbuf, nl.at[pl.ds(lo, SLOTS)])

    @pl.when(is_last)
    def _():
        pltpu.sync_copy(mem_label.at[pl.ds(lo, LAST)], lcbuf.at[pl.ds(0, LAST)])
        pltpu.sync_copy(lcbuf.at[pl.ds(0, LAST)], nl.at[pl.ds(lo, LAST)])

    def move_rows(cnt, dstl, srcl, src_hbm, dstrow, dstlbl, local_vals):
        @pl.when(cnt > 0)
        def _():
            dstl[pl.ds(cnt, 16)] = bcast_at(dstl, cnt - 1)
            srcl[pl.ds(cnt, 16)] = bcast_at(srcl, cnt - 1)
            nch = (cnt + 15) // 16
            npair = (nch + 1) // 2

            def prep(c):
                bb = c * 16
                dv = dstl[pl.ds(bb, 16)]
                sv = srcl[pl.ds(bb, 16)]
                valid = (bb + iota) < cnt
                return (jnp.where(valid, dv, bcast_at(dstl, bb)),
                        jnp.where(valid, sv, bcast_at(srcl, bb)))

            def vals(sv):
                if local_vals:
                    return plsc.load_gather(lcbuf, [sv - lo])
                return plsc.load_gather(ybuf, [sv])

            def pair(k, carry):
                dA, sA = prep(2 * k)
                dB, sB = prep(2 * k + 1)
                idxs[0, :] = sA
                idxs[1, :] = sB
                idxs[2, :] = dA
                idxs[3, :] = dB
                gA = pltpu.async_copy(src_hbm.at[idxs.at[0]],
                                      rowbuf.at[pl.ds(0, 16)], sem_g0)
                gB = pltpu.async_copy(src_hbm.at[idxs.at[1]],
                                      rowbuf.at[pl.ds(16, 16)], sem_g1)
                lbuf[pl.ds(0, 16)] = vals(sA)
                lbuf[pl.ds(16, 16)] = vals(sB)
                lA = pltpu.async_copy(lbuf.at[pl.ds(0, 16)],
                                      dstlbl.at[idxs.at[2]], sem_l0)
                lB = pltpu.async_copy(lbuf.at[pl.ds(16, 16)],
                                      dstlbl.at[idxs.at[3]], sem_l1)
                gA.wait()
                rA = pltpu.async_copy(rowbuf.at[pl.ds(0, 16)],
                                      dstrow.at[idxs.at[2]], sem_s0)
                gB.wait()
                rB = pltpu.async_copy(rowbuf.at[pl.ds(16, 16)],
                                      dstrow.at[idxs.at[3]], sem_s1)
                rA.wait()
                rB.wait()
                lA.wait()
                lB.wait()
                return carry
            lax.fori_loop(0, npair, pair, 0)

    def b_scan(v, cnt):
        pv = poslocal[pl.ds(v * 16, 16)]
        mask = pv >= 0
        slot_abs = lo + v * 16 + iota
        plsc.store_compressed(wslot.at[pl.ds(cnt, 16)], slot_abs, mask=mask)
        plsc.store_compressed(wpos.at[pl.ds(cnt, 16)], pv, mask=mask)
        return cnt + jnp.sum(mask.astype(jnp.int32))
    wcnt = lax.fori_loop(0, myslots // 16, b_scan, _i32(0))
    move_rows(wcnt, wslot, wpos, x, nm, nl, local_vals=False)

    for seg in range(NSEG):
        pltpu.sync_copy(retrieve_idx.at[pl.ds(seg * IB, IB)], ibuf)

        def c_scan(v, carry):
            ucnt, vcnt = carry
            rv = ibuf[pl.ds(v * 16, 16)]
            inr = (rv >= lo) & (rv < hi)
            rloc = jnp.clip(rv - lo, 0, SLOTS - 1)
            pvals = plsc.load_gather(poslocal, [rloc])
            wr = inr & (pvals >= 0)
            un = inr & (pvals < 0)
            j = seg * IB + v * 16 + iota
            plsc.store_compressed(uj.at[pl.ds(ucnt, 16)], j, mask=un)
            plsc.store_compressed(ur.at[pl.ds(ucnt, 16)], rv, mask=un)
            plsc.store_compressed(wj.at[pl.ds(vcnt, 16)], j, mask=wr)
            plsc.store_compressed(wp.at[pl.ds(vcnt, 16)], pvals, mask=wr)
            return (ucnt + jnp.sum(un.astype(jnp.int32)),
                    vcnt + jnp.sum(wr.astype(jnp.int32)))
        ucnt, vcnt = lax.fori_loop(0, IB // 16, c_scan, (_i32(0), _i32(0)))
        move_rows(ucnt, uj, ur, mem, rx, ry, local_vals=True)
        move_rows(vcnt, wj, wp, x, rx, ry, local_vals=False)


def kernel(mem, mem_label, x, y, idx, retrieve_idx):
    return tuple(_sc_kernel(mem, mem_label, x, y, idx, retrieve_idx))


# speedup vs baseline: 1.2811x; 1.0160x over previous
import functools

import jax
import jax.numpy as jnp
from jax import lax
from jax.experimental import pallas as pl
from jax.experimental.pallas import tpu as pltpu
from jax.experimental.pallas import tpu_sc as plsc

M, D, B, NCLS = 100000, 512, 16384, 1000

NC, NS = 2, 16
NW = NC * NS
SLOTS = 3136
LAST = M - (NW - 1) * SLOTS
IB = 8192
NSEG = B // IB
assert SLOTS % 32 == 0 and LAST % 32 == 0 and LAST > 0 and B % IB == 0
_mesh = plsc.VectorSubcoreMesh(core_axis_name="c", subcore_axis_name="s")


def _i32(v):
    return jnp.asarray(v, jnp.int32)


@functools.partial(
    pl.kernel,
    out_type=[
        jax.ShapeDtypeStruct((B, D), jnp.float32),
        jax.ShapeDtypeStruct((B,), jnp.int32),
        jax.ShapeDtypeStruct((M, D), jnp.float32),
        jax.ShapeDtypeStruct((M,), jnp.int32),
    ],
    mesh=_mesh,
    compiler_params=pltpu.CompilerParams(needs_layout_passes=False),
    scratch_types=[
        pltpu.VMEM((IB,), jnp.int32),
        pltpu.VMEM((B,), jnp.int32),
        pltpu.VMEM((SLOTS,), jnp.int32),
        pltpu.VMEM((SLOTS + 48,), jnp.int32),
        pltpu.VMEM((SLOTS + 48,), jnp.int32),
        pltpu.VMEM((IB + 48,), jnp.int32),
        pltpu.VMEM((IB + 48,), jnp.int32),
        pltpu.VMEM((IB + 48,), jnp.int32),
        pltpu.VMEM((IB + 48,), jnp.int32),
        pltpu.VMEM((32, D), jnp.float32),
        pltpu.VMEM((32,), jnp.int32),
        pltpu.VMEM((32, D), jnp.float32),
        pltpu.VMEM((SLOTS,), jnp.int32),
        pltpu.VMEM((IB + 48,), jnp.int32),
        pltpu.SemaphoreType.DMA,
        pltpu.SemaphoreType.DMA,
        pltpu.SemaphoreType.DMA,
        pltpu.SemaphoreType.DMA,
        pltpu.SemaphoreType.DMA,
        pltpu.SemaphoreType.DMA,
        pltpu.SemaphoreType.DMA,
        pltpu.SemaphoreType.DMA,
        pltpu.SemaphoreType.DMA,
        pltpu.SemaphoreType.DMA,
    ],
)
def _sc_kernel(mem, mem_label, x, y, idx, retrieve_idx,
               rx, ry, nm, nl,
               ibuf, ybuf, poslocal, wslot, wpos, uj, ur, wj, wp,
               rowbuf, lbuf, cbuf, lcbuf, lvl,
               sem_g0, sem_g1, sem_s0, sem_s1, sem_l0, sem_l1,
               sem_ia, sem_ib, sem_oa, sem_ob):
    cid = lax.axis_index("c")
    sid = lax.axis_index("s")
    wid = sid * NC + cid
    lo = wid * SLOTS
    is_last = wid == NW - 1
    myslots = jnp.where(is_last, _i32(LAST), _i32(SLOTS))
    hi = lo + myslots
    iota = lax.iota(jnp.int32, 16)
    zeros = iota * 0

    def bcast_at(ref, base):
        return plsc.load_gather(ref, [zeros + base])

    pltpu.sync_copy(y, ybuf)
    neg1 = jnp.full((16,), -1, jnp.int32)

    def init_body(v, carry):
        poslocal[pl.ds(v * 16, 16)] = neg1
        return carry
    lax.fori_loop(0, SLOTS // 16, init_body, 0, unroll=4)

    for seg in range(NSEG):
        pltpu.sync_copy(idx.at[pl.ds(seg * IB, IB)], ibuf)

        def a_body(v, carry):
            idxv = ibuf[pl.ds(v * 16, 16)]
            posv = seg * IB + v * 16 + iota
            inr = (idxv >= lo) & (idxv < hi)
            tgt = jnp.clip(idxv - lo, 0, SLOTS - 1)
            plsc.store_scatter(poslocal, [tgt], posv, mask=inr)

            def w_cond(it):
                cur = plsc.load_gather(poslocal, [tgt])
                return jnp.any(inr & (cur < posv))

            def w_body(it):
                cur = plsc.load_gather(poslocal, [tgt])
                plsc.store_scatter(poslocal, [tgt], posv,
                                   mask=inr & (cur < posv))
                return it + 1
            lax.while_loop(w_cond, w_body, 0)
            return carry
        lax.fori_loop(0, IB // 16, a_body, 0)

    nch2 = myslots // 32

    def in_cp(chunk, half):
        return pltpu.async_copy(
            mem.at[pl.ds(lo + chunk * 16, 16)],
            cbuf.at[pl.ds(half * 16, 16)],
            sem_ia if half == 0 else sem_ib)

    def out_cp(chunk, half):
        return pltpu.async_copy(
            cbuf.at[pl.ds(half * 16, 16)],
            nm.at[pl.ds(lo + chunk * 16, 16)],
            sem_oa if half == 0 else sem_ob)

    in_cp(0, 0)

    def d_body(k, carry):
        ca = 2 * k
        cb = 2 * k + 1
        in_cp(cb, 1)
        pltpu.make_async_copy(mem.at[pl.ds(lo + ca * 16, 16)],
                              cbuf.at[pl.ds(0, 16)], sem_ia).wait()
        oa = out_cp(ca, 0)
        pltpu.make_async_copy(mem.at[pl.ds(lo + cb * 16, 16)],
                              cbuf.at[pl.ds(16, 16)], sem_ib).wait()
        ob = out_cp(cb, 1)
        oa.wait()

        @pl.when(k + 1 < nch2)
        def _():
            in_cp(2 * (k + 1), 0)
        ob.wait()
        return carry
    lax.fori_loop(0, nch2, d_body, 0)

    @pl.when(jnp.logical_not(is_last))
    def _():
        pltpu.sync_copy(mem_label.at[pl.ds(lo, SLOTS)], lcbuf)
        pltpu.sync_copy(lcbuf, nl.at[pl.ds(lo, SLOTS)])

    @pl.when(is_last)
    def _():
        pltpu.sync_copy(mem_label.at[pl.ds(lo, LAST)], lcbuf.at[pl.ds(0, LAST)])
        pltpu.sync_copy(lcbuf.at[pl.ds(0, LAST)], nl.at[pl.ds(lo, LAST)])

    def move_rows(cnt, dstl, srcl, src_hbm, dstrow, dstlbl, local_vals):
        @pl.when(cnt > 0)
        def _():
            dstl[pl.ds(cnt, 16)] = bcast_at(dstl, cnt - 1)
            srcl[pl.ds(cnt, 16)] = bcast_at(srcl, cnt - 1)
            nch = (cnt + 15) // 16
            npair = (nch + 1) // 2

            def repack(c, carry):
                bb = c * 16
                valid = (bb + iota) < cnt
                dv = dstl[pl.ds(bb, 16)]
                sv = srcl[pl.ds(bb, 16)]
                dv = jnp.where(valid, dv, bcast_at(dstl, bb))
                sv = jnp.where(valid, sv, bcast_at(srcl, bb))
                dstl[pl.ds(bb, 16)] = dv
                srcl[pl.ds(bb, 16)] = sv
                if local_vals:
                    lvl[pl.ds(bb, 16)] = plsc.load_gather(lcbuf, [sv - lo])
                else:
                    lvl[pl.ds(bb, 16)] = plsc.load_gather(ybuf, [sv])
                return carry
            lax.fori_loop(0, 2 * npair, repack, 0)

            def pair(k, carry):
                bA = 32 * k
                bB = 32 * k + 16
                gA = pltpu.async_copy(src_hbm.at[srcl.at[pl.ds(bA, 16)]],
                                      rowbuf.at[pl.ds(0, 16)], sem_g0)
                gB = pltpu.async_copy(src_hbm.at[srcl.at[pl.ds(bB, 16)]],
                                      rowbuf.at[pl.ds(16, 16)], sem_g1)
                lA = pltpu.async_copy(lvl.at[pl.ds(bA, 16)],
                                      dstlbl.at[dstl.at[pl.ds(bA, 16)]], sem_l0)
                lB = pltpu.async_copy(lvl.at[pl.ds(bB, 16)],
                                      dstlbl.at[dstl.at[pl.ds(bB, 16)]], sem_l1)
                gA.wait()
                rA = pltpu.async_copy(rowbuf.at[pl.ds(0, 16)],
                                      dstrow.at[dstl.at[pl.ds(bA, 16)]], sem_s0)
                gB.wait()
                rB = pltpu.async_copy(rowbuf.at[pl.ds(16, 16)],
                                      dstrow.at[dstl.at[pl.ds(bB, 16)]], sem_s1)
                rA.wait()
                rB.wait()
                lA.wait()
                lB.wait()
                return carry
            lax.fori_loop(0, npair, pair, 0)

    def b_scan(v, cnt):
        pv = poslocal[pl.ds(v * 16, 16)]
        mask = pv >= 0
        slot_abs = lo + v * 16 + iota
        plsc.store_compressed(wslot.at[pl.ds(cnt, 16)], slot_abs, mask=mask)
        plsc.store_compressed(wpos.at[pl.ds(cnt, 16)], pv, mask=mask)
        return cnt + jnp.sum(mask.astype(jnp.int32))
    wcnt = lax.fori_loop(0, myslots // 16, b_scan, _i32(0))
    move_rows(wcnt, wslot, wpos, x, nm, nl, local_vals=False)

    for seg in range(NSEG):
        pltpu.sync_copy(retrieve_idx.at[pl.ds(seg * IB, IB)], ibuf)

        def c_scan(v, carry):
            ucnt, vcnt = carry
            rv = ibuf[pl.ds(v * 16, 16)]
            inr = (rv >= lo) & (rv < hi)
            rloc = jnp.clip(rv - lo, 0, SLOTS - 1)
            pvals = plsc.load_gather(poslocal, [rloc])
            wr = inr & (pvals >= 0)
            un = inr & (pvals < 0)
            j = seg * IB + v * 16 + iota
            plsc.store_compressed(uj.at[pl.ds(ucnt, 16)], j, mask=un)
            plsc.store_compressed(ur.at[pl.ds(ucnt, 16)], rv, mask=un)
            plsc.store_compressed(wj.at[pl.ds(vcnt, 16)], j, mask=wr)
            plsc.store_compressed(wp.at[pl.ds(vcnt, 16)], pvals, mask=wr)
            return (ucnt + jnp.sum(un.astype(jnp.int32)),
                    vcnt + jnp.sum(wr.astype(jnp.int32)))
        ucnt, vcnt = lax.fori_loop(0, IB // 16, c_scan, (_i32(0), _i32(0)))
        move_rows(ucnt, uj, ur, mem, rx, ry, local_vals=True)
        move_rows(vcnt, wj, wp, x, rx, ry, local_vals=False)


def kernel(mem, mem_label, x, y, idx, retrieve_idx):
    return tuple(_sc_kernel(mem, mem_label, x, y, idx, retrieve_idx))


# speedup vs baseline: 1.5528x; 1.2120x over previous
import functools

import jax
import jax.numpy as jnp
from jax import lax
from jax.experimental import pallas as pl
from jax.experimental.pallas import tpu as pltpu
from jax.experimental.pallas import tpu_sc as plsc

M, D, B, NCLS = 100000, 512, 16384, 1000

NC, NS = 2, 16
NW = NC * NS
SLOTS = 3136
LAST = M - (NW - 1) * SLOTS
assert SLOTS % 32 == 0 and LAST % 32 == 0 and LAST > 0
_mesh = plsc.VectorSubcoreMesh(core_axis_name="c", subcore_axis_name="s")

U_SH, U_MASK = 17, (1 << 17) - 1
W_SH, W_MASK = 14, (1 << 14) - 1


def _i32(v):
    return jnp.asarray(v, jnp.int32)


@functools.partial(
    pl.kernel,
    out_type=[
        jax.ShapeDtypeStruct((B, D), jnp.float32),
        jax.ShapeDtypeStruct((B,), jnp.int32),
        jax.ShapeDtypeStruct((M, D), jnp.float32),
        jax.ShapeDtypeStruct((M,), jnp.int32),
    ],
    mesh=_mesh,
    compiler_params=pltpu.CompilerParams(needs_layout_passes=False),
    scratch_types=[
        pltpu.VMEM((B,), jnp.int32),
        pltpu.VMEM((SLOTS,), jnp.int32),
        pltpu.VMEM((SLOTS + 16,), jnp.int32),
        pltpu.VMEM((SLOTS + 16,), jnp.int32),
        pltpu.VMEM((B + 16,), jnp.int32),
        pltpu.VMEM((B + 16,), jnp.int32),
        pltpu.VMEM((16, D), jnp.float32),
        pltpu.VMEM((16,), jnp.int32),
        pltpu.VMEM((64, D), jnp.float32),
        pltpu.VMEM((SLOTS,), jnp.int32),
        pltpu.SemaphoreType.DMA,
        pltpu.SemaphoreType.DMA,
        pltpu.SemaphoreType.DMA,
        pltpu.SemaphoreType.DMA,
        pltpu.SemaphoreType.DMA,
        pltpu.SemaphoreType.DMA,
        pltpu.SemaphoreType.DMA,
        pltpu.SemaphoreType.DMA,
        pltpu.SemaphoreType.DMA,
        pltpu.SemaphoreType.DMA,
    ],
)
def _sc_kernel(mem, mem_label, x, y, idx, retrieve_idx,
               rx, ry, nm, nl,
               idxbuf, poslocal, wslot, wpos, upk, wpk,
               rowbuf, lbuf, cbuf, lcbuf,
               sem_g, sem_s,
               sem_i0, sem_i1, sem_i2, sem_i3,
               sem_o0, sem_o1, sem_o2, sem_o3):
    cid = lax.axis_index("c")
    sid = lax.axis_index("s")
    wid = sid * NC + cid
    lo = wid * SLOTS
    is_last = wid == NW - 1
    myslots = jnp.where(is_last, _i32(LAST), _i32(SLOTS))
    hi = lo + myslots
    iota = lax.iota(jnp.int32, 16)
    zeros = iota * 0

    def bcast_at(ref, base):
        return plsc.load_gather(ref, [zeros + base])

    pltpu.sync_copy(idx, idxbuf)
    neg1 = jnp.full((16,), -1, jnp.int32)

    def init_body(v, carry):
        poslocal[pl.ds(v * 16, 16)] = neg1
        return carry
    lax.fori_loop(0, SLOTS // 16, init_body, 0, unroll=4)

    def a_body(v, carry):
        idxv = idxbuf[pl.ds(v * 16, 16)]
        posv = v * 16 + iota
        inr = (idxv >= lo) & (idxv < hi)
        tgt = jnp.clip(idxv - lo, 0, SLOTS - 1)
        plsc.store_scatter(poslocal, [tgt], posv, mask=inr)

        def w_cond(it):
            cur = plsc.load_gather(poslocal, [tgt])
            return jnp.any(inr & (cur < posv))

        def w_body(it):
            cur = plsc.load_gather(poslocal, [tgt])
            plsc.store_scatter(poslocal, [tgt], posv, mask=inr & (cur < posv))
            return it + 1
        lax.while_loop(w_cond, w_body, 0)
        return carry
    lax.fori_loop(0, B // 16, a_body, 0)

    nch = myslots // 16

    def in_cp(chunk, q, sem):
        pltpu.async_copy(mem.at[pl.ds(lo + chunk * 16, 16)],
                         cbuf.at[pl.ds(q * 16, 16)], sem)

    def out_cp(chunk, q, sem):
        pltpu.async_copy(cbuf.at[pl.ds(q * 16, 16)],
                         nm.at[pl.ds(lo + chunk * 16, 16)], sem)

    def wait_in(sem):
        pltpu.make_async_copy(mem.at[pl.ds(0, 16)],
                              cbuf.at[pl.ds(0, 16)], sem).wait()

    def wait_out(sem):
        pltpu.make_async_copy(mem.at[pl.ds(0, 16)],
                              cbuf.at[pl.ds(0, 16)], sem).wait()

    in_cp(0, 0, sem_i0)
    in_cp(1, 1, sem_i1)
    in_cp(2, 2, sem_i2)

    def quad(k, carry):
        c = 4 * k
        in_cp(c + 3, 3, sem_i3)
        wait_in(sem_i0)
        out_cp(c, 0, sem_o0)
        wait_in(sem_i1)
        out_cp(c + 1, 1, sem_o1)
        wait_in(sem_i2)
        out_cp(c + 2, 2, sem_o2)
        wait_in(sem_i3)
        out_cp(c + 3, 3, sem_o3)
        wait_out(sem_o0)

        @pl.when(c + 4 < nch)
        def _():
            in_cp(c + 4, 0, sem_i0)
        wait_out(sem_o1)

        @pl.when(c + 5 < nch)
        def _():
            in_cp(c + 5, 1, sem_i1)
        wait_out(sem_o2)

        @pl.when(c + 6 < nch)
        def _():
            in_cp(c + 6, 2, sem_i2)
        wait_out(sem_o3)
        return carry
    lax.fori_loop(0, nch // 4, quad, 0)

    @pl.when(is_last)
    def _():
        wait_in(sem_i0)
        out_cp(LAST // 16 - 2, 0, sem_o0)
        wait_in(sem_i1)
        out_cp(LAST // 16 - 1, 1, sem_o1)
        wait_out(sem_o0)
        wait_out(sem_o1)

    @pl.when(jnp.logical_not(is_last))
    def _():
        pltpu.sync_copy(mem_label.at[pl.ds(lo, SLOTS)], lcbuf)
        pltpu.sync_copy(lcbuf, nl.at[pl.ds(lo, SLOTS)])

    @pl.when(is_last)
    def _():
        pltpu.sync_copy(mem_label.at[pl.ds(lo, LAST)], lcbuf.at[pl.ds(0, LAST)])
        pltpu.sync_copy(lcbuf.at[pl.ds(0, LAST)], nl.at[pl.ds(lo, LAST)])

    def b_scan(v, cnt):
        pv = poslocal[pl.ds(v * 16, 16)]
        mask = pv >= 0
        slot_abs = lo + v * 16 + iota
        plsc.store_compressed(wslot.at[pl.ds(cnt, 16)], slot_abs, mask=mask)
        plsc.store_compressed(wpos.at[pl.ds(cnt, 16)], pv, mask=mask)
        return cnt + jnp.sum(mask.astype(jnp.int32))
    wcnt = lax.fori_loop(0, myslots // 16, b_scan, _i32(0))

    def b_rows(cc, carry):
        base = cc * 16
        tv = wslot[pl.ds(base, 16)]
        pv = wpos[pl.ds(base, 16)]
        valid = (base + iota) < wcnt
        tgt = jnp.where(valid, tv, bcast_at(wslot, base))
        src = jnp.where(valid, pv, bcast_at(wpos, base))
        pltpu.async_copy(x.at[src], rowbuf, sem_g).wait()
        pltpu.async_copy(rowbuf, nm.at[tgt], sem_s).wait()
        pltpu.async_copy(y.at[src], lbuf, sem_g).wait()
        pltpu.async_copy(lbuf, nl.at[tgt], sem_s).wait()
        return carry
    lax.fori_loop(0, (wcnt + 15) // 16, b_rows, 0)

    pltpu.sync_copy(retrieve_idx, idxbuf)

    def c_scan(v, carry):
        ucnt, vcnt = carry
        rv = idxbuf[pl.ds(v * 16, 16)]
        inr = (rv >= lo) & (rv < hi)
        rloc = jnp.clip(rv - lo, 0, SLOTS - 1)
        pvals = plsc.load_gather(poslocal, [rloc])
        wr = inr & (pvals >= 0)
        un = inr & (pvals < 0)
        j = v * 16 + iota
        plsc.store_compressed(upk.at[pl.ds(ucnt, 16)],
                              (j << U_SH) | rv, mask=un)
        plsc.store_compressed(wpk.at[pl.ds(vcnt, 16)],
                              (j << W_SH) | pvals, mask=wr)
        return (ucnt + jnp.sum(un.astype(jnp.int32)),
                vcnt + jnp.sum(wr.astype(jnp.int32)))
    ucnt, vcnt = lax.fori_loop(0, B // 16, c_scan, (_i32(0), _i32(0)))

    def c_unwritten(cc, carry):
        base = cc * 16
        e = upk[pl.ds(base, 16)]
        e = jnp.where((base + iota) < ucnt, e, bcast_at(upk, base))
        jf = lax.shift_right_logical(e, U_SH)
        rf = e & U_MASK
        pltpu.async_copy(mem.at[rf], rowbuf, sem_g).wait()
        pltpu.async_copy(rowbuf, rx.at[jf], sem_s).wait()
        pltpu.async_copy(mem_label.at[rf], lbuf, sem_g).wait()
        pltpu.async_copy(lbuf, ry.at[jf], sem_s).wait()
        return carry
    lax.fori_loop(0, (ucnt + 15) // 16, c_unwritten, 0)

    def c_written(cc, carry):
        base = cc * 16
        e = wpk[pl.ds(base, 16)]
        e = jnp.where((base + iota) < vcnt, e, bcast_at(wpk, base))
        jf = lax.shift_right_logical(e, W_SH)
        pf = e & W_MASK
        pltpu.async_copy(x.at[pf], rowbuf, sem_g).wait()
        pltpu.async_copy(rowbuf, rx.at[jf], sem_s).wait()
        pltpu.async_copy(y.at[pf], lbuf, sem_g).wait()
        pltpu.async_copy(lbuf, ry.at[jf], sem_s).wait()
        return carry
    lax.fori_loop(0, (vcnt + 15) // 16, c_written, 0)


def kernel(mem, mem_label, x, y, idx, retrieve_idx):
    return tuple(_sc_kernel(mem, mem_label, x, y, idx, retrieve_idx))
